# baseline (speedup 1.0000x reference)
import functools

import jax
import jax.numpy as jnp
from jax import lax
from jax.experimental import pallas as pl
from jax.experimental.pallas import tpu as pltpu
from jax.experimental.pallas import tpu_sc as plsc

_NUM_CLASSES = 1000
_DIM = 999
_DIM_PAD = 1008
_BATCH = 16384

_NC = 2
_NS = 16
_NW = _NC * _NS
_B_PER_W = _BATCH // _NW
_CHUNK = 64
_NCHUNK = _B_PER_W // _CHUNK
_NVEC = 62
_TAIL = _DIM - 16


def _gather_body(y_hbm, cb_hbm, out_hbm, idx_v, bufp, bufd, sem):
    wid = lax.axis_index("s") * _NC + lax.axis_index("c")
    base = wid * _B_PER_W
    pltpu.sync_copy(y_hbm.at[pl.ds(wid * _NCHUNK, _NCHUNK)], idx_v)
    for c in range(_NCHUNK):
        pltpu.async_copy(cb_hbm.at[idx_v.at[c]], bufp, sem).wait()

        def compact_row(r, carry):
            for k in range(_NVEC):
                bufd[r, pl.ds(16 * k, 16)] = bufp[r, pl.ds(16 * k, 16)]
            bufd[r, pl.ds(_TAIL, 16)] = bufp[r, pl.ds(_TAIL, 16)]
            return carry

        lax.fori_loop(0, _CHUNK, compact_row, 0)
        pltpu.sync_copy(bufd, out_hbm.at[pl.ds(base + c * _CHUNK, _CHUNK)])


@jax.jit
def _lookup(y, codebook):
    mesh = plsc.VectorSubcoreMesh(core_axis_name="c", subcore_axis_name="s")
    return pl.kernel(
        _gather_body,
        out_type=jax.ShapeDtypeStruct((_BATCH, _DIM), jnp.float32),
        mesh=mesh,
        scratch_types=[
            pltpu.VMEM((_NCHUNK, _CHUNK), jnp.int32),
            pltpu.VMEM((_CHUNK, _DIM_PAD), jnp.float32),
            pltpu.VMEM((_CHUNK, _DIM), jnp.float32),
            pltpu.SemaphoreType.DMA,
        ],
        compiler_params=pltpu.CompilerParams(use_tc_tiling_on_sc=False),
    )(y, codebook)


def kernel(y, codebook):
    y2 = y.astype(jnp.int32).reshape(_NW * _NCHUNK, _CHUNK)
    cb = jnp.pad(codebook, ((0, 0), (0, _DIM_PAD - _DIM)))
    return _lookup(y2, cb)


# speedup vs baseline: 1.5802x; 1.5802x over previous
import functools

import jax
import jax.numpy as jnp
from jax import lax
from jax.experimental import pallas as pl
from jax.experimental.pallas import tpu as pltpu
from jax.experimental.pallas import tpu_sc as plsc

_NUM_CLASSES = 1000
_DIM = 999
_DIM_PAD = 1008
_BATCH = 16384

_NC = 2
_NS = 16
_NW = _NC * _NS
_B_PER_W = _BATCH // _NW
_CHUNK = 32
_NCHUNK = _B_PER_W // _CHUNK
_NVEC = 62
_TAIL = _DIM - 16


def _gather_body(y_hbm, cb_hbm, out_hbm, idx_v, bufp, bufd, gsem, wsem):
    wid = lax.axis_index("s") * _NC + lax.axis_index("c")
    base = wid * _B_PER_W
    pltpu.sync_copy(y_hbm.at[pl.ds(wid * _NCHUNK, _NCHUNK)], idx_v)

    def start_gather(c, s):
        pltpu.async_copy(cb_hbm.at[idx_v.at[c]], bufp.at[s], gsem.at[s])

    def compact(s):
        @plsc.parallel_loop(0, _CHUNK, unroll=1)
        def compact_row(r):
            for k in range(_NVEC):
                bufd[s, r, pl.ds(16 * k, 16)] = bufp[s, r, pl.ds(16 * k, 16)]
            bufd[s, r, pl.ds(_TAIL, 16)] = bufp[s, r, pl.ds(_TAIL, 16)]

    start_gather(0, 0)
    start_gather(1, 1)

    def chunk_pair(i, carry):
        for b in range(2):
            c = 2 * i + b
            pltpu.make_async_copy(
                cb_hbm.at[idx_v.at[c]], bufp.at[b], gsem.at[b]
            ).wait()

            @pl.when(i > 0)
            def _wait_prev_writeback():
                pltpu.make_async_copy(
                    bufd.at[b], out_hbm.at[pl.ds(base, _CHUNK)], wsem.at[b]
                ).wait()

            compact(b)
            pltpu.async_copy(
                bufd.at[b],
                out_hbm.at[pl.ds(base + c * _CHUNK, _CHUNK)],
                wsem.at[b],
            )

            @pl.when(i < _NCHUNK // 2 - 1)
            def _start_next_gather():
                start_gather(c + 2, b)

        return carry

    lax.fori_loop(0, _NCHUNK // 2, chunk_pair, 0)
    for b in range(2):
        pltpu.make_async_copy(
            bufd.at[b], out_hbm.at[pl.ds(base, _CHUNK)], wsem.at[b]
        ).wait()


@jax.jit
def _lookup(y, codebook):
    mesh = plsc.VectorSubcoreMesh(core_axis_name="c", subcore_axis_name="s")
    return pl.kernel(
        _gather_body,
        out_type=jax.ShapeDtypeStruct((_BATCH, _DIM), jnp.float32),
        mesh=mesh,
        scratch_types=[
            pltpu.VMEM((_NCHUNK, _CHUNK), jnp.int32),
            pltpu.VMEM((2, _CHUNK, _DIM_PAD), jnp.float32),
            pltpu.VMEM((2, _CHUNK, _DIM), jnp.float32),
            pltpu.SemaphoreType.DMA((2,)),
            pltpu.SemaphoreType.DMA((2,)),
        ],
        compiler_params=pltpu.CompilerParams(use_tc_tiling_on_sc=False),
    )(y, codebook)


def kernel(y, codebook):
    y2 = y.astype(jnp.int32).reshape(_NW * _NCHUNK, _CHUNK)
    cb = jnp.pad(codebook, ((0, 0), (0, _DIM_PAD - _DIM)))
    return _lookup(y2, cb)


# speedup vs baseline: 2.2907x; 1.4496x over previous
import functools

import jax
import jax.numpy as jnp
from jax import lax
from jax.experimental import pallas as pl
from jax.experimental.pallas import tpu as pltpu
from jax.experimental.pallas import tpu_sc as plsc

_NUM_CLASSES = 1000
_DIM = 999
_DIM_PAD = 1024
_BATCH = 16384

_NC = 2
_NS = 16
_NW = _NC * _NS
_B_PER_W = _BATCH // _NW
_CHUNK = 16
_NCHUNK = _B_PER_W // _CHUNK
_NVEC = 62
_TAIL = _DIM - 16


def _gather_body(y_hbm, cb_hbm, out_hbm, idx_v, bufp, bufd, gsem, wsem):
    wid = lax.axis_index("s") * _NC + lax.axis_index("c")
    base = wid * _B_PER_W
    pltpu.sync_copy(y_hbm.at[pl.ds(wid * _NCHUNK, _NCHUNK)], idx_v)

    def start_gather(c, s):
        pltpu.async_copy(cb_hbm.at[idx_v.at[c]], bufp.at[s], gsem.at[s])

    def compact(s):
        @plsc.parallel_loop(0, _CHUNK, unroll=1)
        def compact_row(r):
            for k in range(_NVEC):
                bufd[s, r, pl.ds(16 * k, 16)] = bufp[s, r, pl.ds(16 * k, 16)]
            bufd[s, r, pl.ds(_TAIL, 16)] = bufp[s, r, pl.ds(_TAIL, 16)]

    start_gather(0, 0)
    start_gather(1, 1)

    def chunk_pair(i, carry):
        for b in range(2):
            c = 2 * i + b
            pltpu.make_async_copy(
                cb_hbm.at[idx_v.at[c]], bufp.at[b], gsem.at[b]
            ).wait()

            @pl.when(i > 0)
            def _wait_prev_writeback():
                pltpu.make_async_copy(
                    bufd.at[b], out_hbm.at[pl.ds(base, _CHUNK)], wsem.at[b]
                ).wait()

            compact(b)
            pltpu.async_copy(
                bufd.at[b],
                out_hbm.at[pl.ds(base + c * _CHUNK, _CHUNK)],
                wsem.at[b],
            )

            @pl.when(i < _NCHUNK // 2 - 1)
            def _start_next_gather():
                start_gather(c + 2, b)

        return carry

    lax.fori_loop(0, _NCHUNK // 2, chunk_pair, 0)
    for b in range(2):
        pltpu.make_async_copy(
            bufd.at[b], out_hbm.at[pl.ds(base, _CHUNK)], wsem.at[b]
        ).wait()


@jax.jit
def _lookup(y, codebook):
    mesh = plsc.VectorSubcoreMesh(core_axis_name="c", subcore_axis_name="s")
    return pl.kernel(
        _gather_body,
        out_type=jax.ShapeDtypeStruct((_BATCH, _DIM), jnp.float32),
        mesh=mesh,
        scratch_types=[
            pltpu.VMEM((_NCHUNK, _CHUNK), jnp.int32),
            pltpu.VMEM((2, _CHUNK, _DIM_PAD), jnp.float32),
            pltpu.VMEM((2, _CHUNK, _DIM), jnp.float32),
            pltpu.SemaphoreType.DMA((2,)),
            pltpu.SemaphoreType.DMA((2,)),
        ],
        compiler_params=pltpu.CompilerParams(use_tc_tiling_on_sc=True),
    )(y, codebook)


def kernel(y, codebook):
    y2 = y.astype(jnp.int32).reshape(_NW * _NCHUNK, _CHUNK)
    cb = jnp.pad(codebook, ((0, 0), (0, _DIM_PAD - _DIM)))
    return _lookup(y2, cb)
